# baseline (speedup 1.0000x reference)
import functools

import jax
import jax.numpy as jnp
from jax import lax
from jax.experimental import pallas as pl
from jax.experimental.pallas import tpu as pltpu
from jax.experimental.pallas import tpu_sc as plsc


def _sc_gather(inputs, table):
    B = inputs.shape[0]
    V, D = table.shape
    info = plsc.get_sparse_core_info()
    nw = info.num_cores * info.num_subcores
    b_per_w = B // nw
    mesh = plsc.VectorSubcoreMesh(core_axis_name="c", subcore_axis_name="s")

    @functools.partial(
        pl.kernel,
        mesh=mesh,
        out_type=jax.ShapeDtypeStruct((B, D), jnp.float32),
        scratch_types=[
            pltpu.VMEM((b_per_w,), jnp.int32),
            pltpu.VMEM((b_per_w, D), jnp.float32),
            pltpu.SemaphoreType.DMA,
        ],
    )
    def gather_kernel(idx_hbm, table_hbm, out_hbm, idx_v, rows_v, sem):
        wid = lax.axis_index("s") * info.num_cores + lax.axis_index("c")
        base = wid * b_per_w
        pltpu.sync_copy(idx_hbm.at[pl.ds(base, b_per_w)], idx_v)
        pltpu.async_copy(table_hbm.at[idx_v], rows_v, sem).wait()
        pltpu.sync_copy(rows_v, out_hbm.at[pl.ds(base, b_per_w)])

    return gather_kernel(inputs, table)


def _tc_project(emb, W, b, block_m, block_n, nbuf):
    B, D = emb.shape
    V = W.shape[1]
    n_full = V // block_n
    m_blocks = B // block_m
    n_steps = n_full * m_blocks
    b2 = b.reshape(1, V)

    def mm_kernel(emb_ref, w_ref, b_ref, out_hbm, bufs, sems):
        n = pl.program_id(0)
        m = pl.program_id(1)
        j = n * m_blocks + m
        slot = jax.lax.rem(j, nbuf)
        acc = (
            jnp.dot(
                emb_ref[...],
                w_ref[...].astype(jnp.bfloat16),
                preferred_element_type=jnp.float32,
            )
            + b_ref[...]
        )
        for k in range(nbuf):
            @pl.when(jnp.logical_and(slot == k, j >= nbuf))
            def _():
                pltpu.make_async_copy(
                    bufs.at[k],
                    out_hbm.at[pl.ds(0, block_m), pl.ds(0, block_n)],
                    sems.at[k],
                ).wait()

            @pl.when(slot == k)
            def _():
                bufs[k] = acc
                pltpu.make_async_copy(
                    bufs.at[k],
                    out_hbm.at[
                        pl.ds(m * block_m, block_m), pl.ds(n * block_n, block_n)
                    ],
                    sems.at[k],
                ).start()

        @pl.when(j == n_steps - 1)
        def _():
            for k2 in range(min(nbuf, n_steps)):
                pltpu.make_async_copy(
                    bufs.at[k2],
                    out_hbm.at[pl.ds(0, block_m), pl.ds(0, block_n)],
                    sems.at[k2],
                ).wait()

    partial = pl.pallas_call(
        mm_kernel,
        grid=(n_full, m_blocks),
        in_specs=[
            pl.BlockSpec((block_m, D), lambda n, m: (m, 0)),
            pl.BlockSpec((D, block_n), lambda n, m: (0, n)),
            pl.BlockSpec((1, block_n), lambda n, m: (0, n)),
        ],
        out_specs=pl.BlockSpec(memory_space=pl.ANY),
        out_shape=jax.ShapeDtypeStruct((B, V), jnp.float32),
        scratch_shapes=[
            pltpu.VMEM((nbuf, block_m, block_n), jnp.float32),
            pltpu.SemaphoreType.DMA((nbuf,)),
        ],
    )(emb, W, b2)

    if n_full * block_n == V:
        return partial

    def edge_kernel(emb_ref, w_ref, b_ref, full_ref, out_ref):
        del full_ref
        out_ref[...] = (
            jnp.dot(
                emb_ref[...],
                w_ref[...].astype(jnp.bfloat16),
                preferred_element_type=jnp.float32,
            )
            + b_ref[...]
        )

    return pl.pallas_call(
        edge_kernel,
        grid=(m_blocks,),
        in_specs=[
            pl.BlockSpec((block_m, D), lambda m: (m, 0)),
            pl.BlockSpec((D, block_n), lambda m: (0, n_full)),
            pl.BlockSpec((1, block_n), lambda m: (0, n_full)),
            pl.BlockSpec(memory_space=pl.ANY),
        ],
        out_specs=pl.BlockSpec((block_m, block_n), lambda m: (m, n_full)),
        out_shape=jax.ShapeDtypeStruct((B, V), jnp.float32),
        input_output_aliases={3: 0},
    )(emb, W, b2, partial)


def kernel(inputs, table, W, b):
    emb = _sc_gather(inputs, table).astype(jnp.bfloat16)
    return _tc_project(emb, W, b, block_m=2048, block_n=2048, nbuf=2)
